# baseline (speedup 1.0000x reference)
import jax
import jax.numpy as jnp
from jax import lax
from jax.experimental import pallas as pl


def _body(wt_ref, out_ref):
    _, C, Kb, L = out_ref.shape
    E = wt_ref.shape[0]
    PE_C = C - E

    c_i = lax.broadcasted_iota(jnp.int32, (PE_C, L), 0)
    l_f = lax.broadcasted_iota(jnp.int32, (PE_C, L), 1).astype(jnp.float32)
    pair = (c_i // 2).astype(jnp.float32)
    div = jnp.exp(pair * (-jnp.log(10000.0) * 2.0 / PE_C))
    angle = l_f * div
    pe = jnp.where((c_i % 2) == 0, jnp.sin(angle), jnp.cos(angle))

    out_ref[0, 0:PE_C, :, :] = jnp.broadcast_to(pe[:, None, :], (PE_C, Kb, L))
    out_ref[0, PE_C:C, :, :] = jnp.broadcast_to(wt_ref[...], (E, Kb, L))


def kernel(cond_mask, embed_weight):
    B, _, K, L = cond_mask.shape
    E = embed_weight.shape[1]
    C = 128 + E

    KB = 4
    Kb = K // KB
    wt = embed_weight.T[:, :, None]

    return pl.pallas_call(
        _body,
        grid=(B, KB),
        in_specs=[pl.BlockSpec((E, Kb, 1), lambda b, kb: (0, kb, 0))],
        out_specs=pl.BlockSpec((1, C, Kb, L), lambda b, kb: (b, 0, kb, 0)),
        out_shape=jax.ShapeDtypeStruct((B, C, K, L), jnp.float32),
    )(wt)


# speedup vs baseline: 1.1102x; 1.1102x over previous
import jax
import jax.numpy as jnp
from jax import lax
from jax.experimental import pallas as pl


def _body(wt_ref, out_ref):
    _, C, Kb, L = out_ref.shape
    E = wt_ref.shape[0]
    PE_C = C - E

    c_i = lax.broadcasted_iota(jnp.int32, (PE_C, L), 0)
    l_f = lax.broadcasted_iota(jnp.int32, (PE_C, L), 1).astype(jnp.float32)
    pair = (c_i // 2).astype(jnp.float32)
    div = jnp.exp(pair * (-jnp.log(10000.0) * 2.0 / PE_C))
    angle = l_f * div
    pe = jnp.where((c_i % 2) == 0, jnp.sin(angle), jnp.cos(angle))

    out_ref[0, 0:PE_C, :, :] = jnp.broadcast_to(pe[:, None, :], (PE_C, Kb, L))
    out_ref[0, PE_C:C, :, :] = jnp.broadcast_to(wt_ref[...], (E, Kb, L))


def kernel(cond_mask, embed_weight):
    B, _, K, L = cond_mask.shape
    E = embed_weight.shape[1]
    C = 128 + E

    KB = 1
    Kb = K // KB
    wt = embed_weight.T[:, :, None]

    return pl.pallas_call(
        _body,
        grid=(B, KB),
        in_specs=[pl.BlockSpec((E, Kb, 1), lambda b, kb: (0, kb, 0))],
        out_specs=pl.BlockSpec((1, C, Kb, L), lambda b, kb: (b, 0, kb, 0)),
        out_shape=jax.ShapeDtypeStruct((B, C, K, L), jnp.float32),
    )(wt)


# speedup vs baseline: 1.1121x; 1.0017x over previous
import jax
import jax.numpy as jnp
from jax import lax
from jax.experimental import pallas as pl
from jax.experimental.pallas import tpu as pltpu


def _make_body(B):
    def body(wt_ref, out_ref, scratch, sem):
        C, K, L = scratch.shape
        E = wt_ref.shape[0]
        PE_C = C - E

        c_i = lax.broadcasted_iota(jnp.int32, (PE_C, L), 0)
        l_f = lax.broadcasted_iota(jnp.int32, (PE_C, L), 1).astype(jnp.float32)
        pair = (c_i // 2).astype(jnp.float32)
        div = jnp.exp(pair * (-jnp.log(10000.0) * 2.0 / PE_C))
        angle = l_f * div
        pe = jnp.where((c_i % 2) == 0, jnp.sin(angle), jnp.cos(angle))

        scratch[0:PE_C, :, :] = jnp.broadcast_to(pe[:, None, :], (PE_C, K, L))
        scratch[PE_C:C, :, :] = jnp.broadcast_to(wt_ref[...], (E, K, L))

        copies = [
            pltpu.make_async_copy(scratch, out_ref.at[b], sem) for b in range(B)
        ]
        for c in copies:
            c.start()
        for c in copies:
            c.wait()

    return body


def kernel(cond_mask, embed_weight):
    B, _, K, L = cond_mask.shape
    E = embed_weight.shape[1]
    C = 128 + E
    wt = embed_weight.T[:, :, None]

    return pl.pallas_call(
        _make_body(B),
        in_specs=[pl.BlockSpec(memory_space=pltpu.VMEM)],
        out_specs=pl.BlockSpec(memory_space=pl.ANY),
        out_shape=jax.ShapeDtypeStruct((B, C, K, L), jnp.float32),
        scratch_shapes=[
            pltpu.VMEM((C, K, L), jnp.float32),
            pltpu.SemaphoreType.DMA,
        ],
    )(wt)
